# Initial kernel scaffold; baseline (speedup 1.0000x reference)
import jax
import jax.numpy as jnp
from jax.experimental import pallas as pl


def kernel(x, edge_index, W1, as1, ad1, b1, W2, as2, ad2, b2, W3, as3, ad3, b3):
    raise NotImplementedError("write your pallas kernel here")


# speedup vs baseline: 33.5670x; 33.5670x over previous
import functools
import jax
import jax.numpy as jnp
import numpy as np
from jax import lax
from jax.experimental import pallas as pl
from jax.experimental.pallas import tpu as pltpu
from jax.experimental.pallas import tpu_sc as plsc

H = 5
K_EDGE = 64
NSUB = 16
SELU_L = 1.0507009873554805
SELU_A = 1.6732632423543772


def _selu(x):
    return SELU_L * jnp.where(x > 0, x, SELU_A * (jnp.exp(x) - 1.0))


def _mm_body(x_ref, w_ref, o_ref):
    o_ref[...] = jnp.dot(x_ref[...], w_ref[...], preferred_element_type=jnp.float32)


def _mm(x, w, br=1024):
    npad, kd = x.shape
    m = w.shape[1]
    return pl.pallas_call(
        _mm_body,
        grid=(npad // br,),
        in_specs=[
            pl.BlockSpec((br, kd), lambda i: (i, 0)),
            pl.BlockSpec((kd, m), lambda i: (0, 0)),
        ],
        out_specs=pl.BlockSpec((br, m), lambda i: (i, 0)),
        out_shape=jax.ShapeDtypeStruct((npad, m), jnp.float32),
    )(x, w)


def _div_selu_mm_body(num_ref, den_ref, bmat_ref, brow_ref, w_ref, o_ref):
    dexp = jnp.dot(den_ref[...], bmat_ref[...], preferred_element_type=jnp.float32)
    hin = num_ref[...] / (dexp + 1e-16) + brow_ref[0:1, :]
    hin = _selu(hin)
    o_ref[...] = jnp.dot(hin, w_ref[...], preferred_element_type=jnp.float32)


def _div_selu_mm(num, den, bmat, brow, w, br=1024):
    npad, m_in = num.shape
    m_out = w.shape[1]
    return pl.pallas_call(
        _div_selu_mm_body,
        grid=(npad // br,),
        in_specs=[
            pl.BlockSpec((br, m_in), lambda i: (i, 0)),
            pl.BlockSpec((br, 128), lambda i: (i, 0)),
            pl.BlockSpec((128, m_in), lambda i: (0, 0)),
            pl.BlockSpec((8, m_in), lambda i: (0, 0)),
            pl.BlockSpec((m_in, m_out), lambda i: (0, 0)),
        ],
        out_specs=pl.BlockSpec((br, m_out), lambda i: (i, 0)),
        out_shape=jax.ShapeDtypeStruct((npad, m_out), jnp.float32),
    )(num, den, bmat, brow, w)


def _final_body(num_ref, den_ref, bmat_ref, brow_ref, o_ref):
    dexp = jnp.dot(den_ref[...], bmat_ref[...], preferred_element_type=jnp.float32)
    h = num_ref[...] / (dexp + 1e-16) + brow_ref[0:1, :]
    h = jnp.clip(h, 0.0, 1.0) ** 2
    nrm = jnp.sqrt(jnp.sum(h * h, axis=1, keepdims=True))
    o_ref[...] = h / jnp.maximum(nrm, 1e-12)


def _final(num, den, bmat, brow, br=1024):
    npad, m_in = num.shape
    return pl.pallas_call(
        _final_body,
        grid=(npad // br,),
        in_specs=[
            pl.BlockSpec((br, m_in), lambda i: (i, 0)),
            pl.BlockSpec((br, 128), lambda i: (i, 0)),
            pl.BlockSpec((128, m_in), lambda i: (0, 0)),
            pl.BlockSpec((8, m_in), lambda i: (0, 0)),
        ],
        out_specs=pl.BlockSpec((br, m_in), lambda i: (i, 0)),
        out_shape=jax.ShapeDtypeStruct((npad, m_in), jnp.float32),
    )(num, den, bmat, brow)


def _sc_edge(htab, aux_s2, aux_d, srcr, dstr, *, cw, c_head, nacc, ch2):
    nt8 = cw // 16
    cph = c_head // 16
    rpt = nacc // NSUB
    mesh = plsc.VectorSubcoreMesh(core_axis_name="c", subcore_axis_name="s")

    @functools.partial(
        pl.kernel,
        out_type=(
            jax.ShapeDtypeStruct((2, nacc, cw), jnp.float32),
            jax.ShapeDtypeStruct((nacc, 16), jnp.float32),
        ),
        mesh=mesh,
        compiler_params=pltpu.CompilerParams(use_tc_tiling_on_sc=False),
        scratch_types=[
            pltpu.VMEM((K_EDGE,), jnp.int32),
            pltpu.VMEM((K_EDGE,), jnp.int32),
            pltpu.VMEM((K_EDGE,), jnp.int32),
            pltpu.VMEM((K_EDGE, cw), jnp.float32),
            pltpu.VMEM((K_EDGE, 16), jnp.float32),
            pltpu.VMEM((K_EDGE, 16), jnp.float32),
            pltpu.VMEM((K_EDGE, 16), jnp.float32),
            pltpu.VMEM_SHARED((nacc, cw), jnp.float32),
            pltpu.VMEM_SHARED((nacc, 16), jnp.float32),
            pltpu.SemaphoreType.DMA,
            pltpu.SemaphoreType.DMA,
            pltpu.SemaphoreType.DMA,
        ],
    )
    def kfn(htab_r, auxs_r, auxd_r, srcr_r, dstr_r,
            num_o, den_o,
            srcb, hsrcb, dstb, hrows, srows, drows, wbuf,
            num_sh, den_sh, s1, s2, s3):
        c = lax.axis_index("c")
        s = lax.axis_index("s")
        r0 = s * rpt
        zv = jnp.zeros((16,), jnp.float32)

        def zr(k, _):
            for t8 in range(nt8):
                hrows[k, pl.ds(t8 * 16, 16)] = zv
            wbuf[k] = zv
            return 0

        lax.fori_loop(0, K_EDGE, zr, 0)
        nfull = rpt // K_EDGE
        for q in range(nfull):
            pltpu.sync_copy(hrows, num_sh.at[pl.ds(r0 + q * K_EDGE, K_EDGE)])
            pltpu.sync_copy(wbuf, den_sh.at[pl.ds(r0 + q * K_EDGE, K_EDGE)])
        tail = rpt - nfull * K_EDGE
        if tail:
            pltpu.sync_copy(hrows.at[pl.ds(0, tail)],
                            num_sh.at[pl.ds(r0 + nfull * K_EDGE, tail)])
            pltpu.sync_copy(wbuf.at[pl.ds(0, tail)],
                            den_sh.at[pl.ds(r0 + nfull * K_EDGE, tail)])
        off = c * nacc
        plsc.subcore_barrier()

        def chunk(j, _):
            pltpu.sync_copy(srcr_r.at[s, j], srcb)
            pltpu.sync_copy(dstr_r.at[s, j], dstb)
            for t in range(K_EDGE // 16):
                hsrcb[pl.ds(t * 16, 16)] = srcb[pl.ds(t * 16, 16)] + off
            cp1 = pltpu.async_copy(htab_r.at[hsrcb], hrows, s1)
            cp2 = pltpu.async_copy(auxs_r.at[hsrcb], srows, s2)
            cp3 = pltpu.async_copy(auxd_r.at[dstb], drows, s3)
            cp2.wait()
            cp3.wait()
            cp1.wait()

            def edge(k, _):
                t = srows[k] + drows[k]
                w = jnp.exp(jnp.maximum(t, 0.2 * t))
                wbuf[k] = w
                for t8 in range(nt8):
                    head = (c * nt8 + t8) // cph
                    hidx = jnp.full((16,), head, dtype=jnp.int32)
                    ws = lax.gather(
                        w, hidx[:, None],
                        dimension_numbers=lax.GatherDimensionNumbers(
                            offset_dims=(), collapsed_slice_dims=(0,),
                            start_index_map=(0,)),
                        slice_sizes=(1,),
                        mode=lax.GatherScatterMode.PROMISE_IN_BOUNDS)
                    hrows[k, pl.ds(t8 * 16, 16)] = hrows[k, pl.ds(t8 * 16, 16)] * ws
                return 0

            lax.fori_loop(0, K_EDGE, edge, 0)
            pltpu.sync_copy(hrows, num_sh.at[dstb], add=True)

            @pl.when(c == 0)
            def _():
                pltpu.sync_copy(wbuf, den_sh.at[dstb], add=True)

            return 0

        lax.fori_loop(0, ch2, chunk, 0)
        plsc.subcore_barrier()
        pltpu.sync_copy(num_sh.at[pl.ds(r0, rpt)], num_o.at[c, pl.ds(r0, rpt)])

        @pl.when(c == 0)
        def _():
            pltpu.sync_copy(den_sh.at[pl.ds(r0, rpt)], den_o.at[pl.ds(r0, rpt)])

    return kfn(htab, aux_s2, aux_d, srcr, dstr)


def _fold_attn(w, a_s, a_d):
    d, m = w.shape
    c = m // H
    wr = w.reshape(d, H, c)
    ws = jnp.einsum('dhc,hc->dh', wr, a_s)
    wd = jnp.einsum('dhc,hc->dh', wr, a_d)
    return jnp.concatenate([w, ws, wd], axis=1)


def _pad2(a, rows, cols):
    return jnp.pad(a, ((0, rows - a.shape[0]), (0, cols - a.shape[1])))


def _head_expand_mat(c, m_in):
    b = np.zeros((128, m_in), np.float32)
    for h in range(H):
        b[h, h * c:(h + 1) * c] = 1.0
    return jnp.asarray(b)


def _layer_tables(hcat, crow, cw, n, nacc):
    h = hcat[:n, :crow]
    h0 = h[:, :cw]
    h1 = _pad2(h[:, cw:crow], n, cw)
    htab = jnp.concatenate([_pad2(h0, nacc, cw), _pad2(h1, nacc, cw)], axis=0)
    aux_s = _pad2(hcat[:n, crow:crow + H], nacc, 16)
    aux_d = _pad2(hcat[:n, crow + H:crow + 2 * H], nacc, 16)
    aux_s2 = jnp.concatenate([aux_s, aux_s], axis=0)
    return htab, aux_s2, aux_d


def kernel(x, edge_index, W1, as1, ad1, b1, W2, as2, ad2, b2, W3, as3, ad3, b3):
    n, d = x.shape
    e = edge_index.shape[1]
    npad = ((n + 1023) // 1024) * 1024
    nacc = ((n + 8 + NSUB * 8 - 1) // (NSUB * 8)) * (NSUB * 8)

    loops = jnp.arange(n, dtype=jnp.int32)
    src = jnp.concatenate([edge_index[0].astype(jnp.int32), loops])
    dst = jnp.concatenate([edge_index[1].astype(jnp.int32), loops])
    etot = e + n
    ch2 = -(-etot // (NSUB * K_EDGE))
    ept = NSUB * ch2 * K_EDGE
    src = jnp.pad(src, (0, ept - etot), constant_values=n).reshape(NSUB, ch2, K_EDGE)
    dst = jnp.pad(dst, (0, ept - etot), constant_values=n).reshape(NSUB, ch2, K_EDGE)

    c1, c2, c3 = W1.shape[1] // H, W2.shape[1] // H, W3.shape[1] // H
    crow1, crow2, crow3 = H * c1, H * c2, H * c3
    cw1, cw2, cw3 = crow1 // 2, crow2 // 2, 48
    m1p, m2p, m3p = 384, 256, 128

    wa1 = _pad2(_fold_attn(W1, as1, ad1), d, m1p)
    wa2 = _pad2(_fold_attn(W2, as2, ad2), m1p, m2p)
    wa3 = _pad2(_fold_attn(W3, as3, ad3), m2p, m3p)
    bm2 = _head_expand_mat(c1, m1p)
    bm3 = _head_expand_mat(c2, m2p)
    bm4 = _head_expand_mat(c3, m3p)
    b1r = jnp.tile(jnp.pad(b1, (0, m1p - crow1))[None, :], (8, 1))
    b2r = jnp.tile(jnp.pad(b2, (0, m2p - crow2))[None, :], (8, 1))
    b3r = jnp.tile(jnp.pad(b3, (0, m3p - crow3))[None, :], (8, 1))

    xp = jnp.pad(x, ((0, npad - n), (0, 0)))

    def edge_pass(hcat, crow, cw, c_head):
        htab, aux_s2, aux_d = _layer_tables(hcat, crow, cw, n, nacc)
        num, den = _sc_edge(htab, aux_s2, aux_d, src, dst,
                            cw=cw, c_head=c_head, nacc=nacc, ch2=ch2)
        num = jnp.concatenate([num[0], num[1]], axis=1)
        return num, den

    hcat1 = _mm(xp, wa1)
    num1, den1 = edge_pass(hcat1, crow1, cw1, c1)
    num1 = _pad2(num1, npad, m1p)
    den1 = _pad2(den1, npad, 128)
    hcat2 = _div_selu_mm(num1, den1, bm2, b1r, wa2)
    num2, den2 = edge_pass(hcat2, crow2, cw2, c2)
    num2 = _pad2(num2, npad, m2p)
    den2 = _pad2(den2, npad, 128)
    hcat3 = _div_selu_mm(num2, den2, bm3, b2r, wa3)
    num3, den3 = edge_pass(hcat3, crow3, cw3, c3)
    num3 = _pad2(num3, npad, m3p)
    den3 = _pad2(den3, npad, 128)
    out = _final(num3, den3, bm4, b3r)
    return out[:n, :crow3]


# speedup vs baseline: 42.4228x; 1.2638x over previous
import functools
import jax
import jax.numpy as jnp
import numpy as np
from jax import lax
from jax.experimental import pallas as pl
from jax.experimental.pallas import tpu as pltpu
from jax.experimental.pallas import tpu_sc as plsc

H = 5
NSUB = 16
SELU_L = 1.0507009873554805
SELU_A = 1.6732632423543772


def _selu(x):
    return SELU_L * jnp.where(x > 0, x, SELU_A * (jnp.exp(x) - 1.0))


def _mm_body(x_ref, w_ref, o_ref):
    o_ref[...] = jnp.dot(x_ref[...], w_ref[...], preferred_element_type=jnp.float32)


def _mm(x, w, br=1024):
    npad, kd = x.shape
    m = w.shape[1]
    return pl.pallas_call(
        _mm_body,
        grid=(npad // br,),
        in_specs=[
            pl.BlockSpec((br, kd), lambda i: (i, 0)),
            pl.BlockSpec((kd, m), lambda i: (0, 0)),
        ],
        out_specs=pl.BlockSpec((br, m), lambda i: (i, 0)),
        out_shape=jax.ShapeDtypeStruct((npad, m), jnp.float32),
    )(x, w)


def _div_selu_mm_body(num_ref, den_ref, bmat_ref, brow_ref, w_ref, o_ref):
    dexp = jnp.dot(den_ref[...], bmat_ref[...], preferred_element_type=jnp.float32)
    hin = num_ref[...] / (dexp + 1e-16) + brow_ref[0:1, :]
    hin = _selu(hin)
    o_ref[...] = jnp.dot(hin, w_ref[...], preferred_element_type=jnp.float32)


def _div_selu_mm(num, den, bmat, brow, w, br=1024):
    npad, m_in = num.shape
    m_out = w.shape[1]
    return pl.pallas_call(
        _div_selu_mm_body,
        grid=(npad // br,),
        in_specs=[
            pl.BlockSpec((br, m_in), lambda i: (i, 0)),
            pl.BlockSpec((br, 128), lambda i: (i, 0)),
            pl.BlockSpec((128, m_in), lambda i: (0, 0)),
            pl.BlockSpec((8, m_in), lambda i: (0, 0)),
            pl.BlockSpec((m_in, m_out), lambda i: (0, 0)),
        ],
        out_specs=pl.BlockSpec((br, m_out), lambda i: (i, 0)),
        out_shape=jax.ShapeDtypeStruct((npad, m_out), jnp.float32),
    )(num, den, bmat, brow, w)


def _final_body(num_ref, den_ref, bmat_ref, brow_ref, o_ref):
    dexp = jnp.dot(den_ref[...], bmat_ref[...], preferred_element_type=jnp.float32)
    h = num_ref[...] / (dexp + 1e-16) + brow_ref[0:1, :]
    h = jnp.clip(h, 0.0, 1.0) ** 2
    nrm = jnp.sqrt(jnp.sum(h * h, axis=1, keepdims=True))
    o_ref[...] = h / jnp.maximum(nrm, 1e-12)


def _final(num, den, bmat, brow, br=1024):
    npad, m_in = num.shape
    return pl.pallas_call(
        _final_body,
        grid=(npad // br,),
        in_specs=[
            pl.BlockSpec((br, m_in), lambda i: (i, 0)),
            pl.BlockSpec((br, 128), lambda i: (i, 0)),
            pl.BlockSpec((128, m_in), lambda i: (0, 0)),
            pl.BlockSpec((8, m_in), lambda i: (0, 0)),
        ],
        out_specs=pl.BlockSpec((br, m_in), lambda i: (i, 0)),
        out_shape=jax.ShapeDtypeStruct((npad, m_in), jnp.float32),
    )(num, den, bmat, brow)


def _sc_edge(htab, aux_s2, aux_d, srcr, dstr, *, cw, c_head, nacc, ch2, k_edge):
    nt8 = cw // 16
    cph = c_head // 16
    rpt = nacc // NSUB
    mesh = plsc.VectorSubcoreMesh(core_axis_name="c", subcore_axis_name="s")

    buf_types = []
    for _ in range(2):
        buf_types += [
            pltpu.VMEM((k_edge,), jnp.int32),
            pltpu.VMEM((k_edge,), jnp.int32),
            pltpu.VMEM((k_edge,), jnp.int32),
            pltpu.VMEM((k_edge, cw), jnp.float32),
            pltpu.VMEM((k_edge, 16), jnp.float32),
            pltpu.VMEM((k_edge, 16), jnp.float32),
            pltpu.VMEM((k_edge, 16), jnp.float32),
            pltpu.SemaphoreType.DMA,
            pltpu.SemaphoreType.DMA,
            pltpu.SemaphoreType.DMA,
        ]

    @functools.partial(
        pl.kernel,
        out_type=(
            jax.ShapeDtypeStruct((2, nacc, cw), jnp.float32),
            jax.ShapeDtypeStruct((nacc, 16), jnp.float32),
        ),
        mesh=mesh,
        compiler_params=pltpu.CompilerParams(use_tc_tiling_on_sc=False),
        scratch_types=buf_types + [
            pltpu.VMEM_SHARED((nacc, cw), jnp.float32),
            pltpu.VMEM_SHARED((nacc, 16), jnp.float32),
        ],
    )
    def kfn(htab_r, auxs_r, auxd_r, srcr_r, dstr_r, num_o, den_o, *scr):
        bufs = [scr[0:10], scr[10:20]]
        num_sh, den_sh = scr[20], scr[21]
        c = lax.axis_index("c")
        s = lax.axis_index("s")
        r0 = s * rpt
        zv = jnp.zeros((16,), jnp.float32)
        hrows0, wbuf0 = bufs[0][3], bufs[0][6]

        def zr(k, _):
            for t8 in range(nt8):
                hrows0[k, pl.ds(t8 * 16, 16)] = zv
            wbuf0[k] = zv
            return 0

        lax.fori_loop(0, k_edge, zr, 0)
        nfull = rpt // k_edge
        for q in range(nfull):
            pltpu.sync_copy(hrows0, num_sh.at[pl.ds(r0 + q * k_edge, k_edge)])
            pltpu.sync_copy(wbuf0, den_sh.at[pl.ds(r0 + q * k_edge, k_edge)])
        tail = rpt - nfull * k_edge
        if tail:
            pltpu.sync_copy(hrows0.at[pl.ds(0, tail)],
                            num_sh.at[pl.ds(r0 + nfull * k_edge, tail)])
            pltpu.sync_copy(wbuf0.at[pl.ds(0, tail)],
                            den_sh.at[pl.ds(r0 + nfull * k_edge, tail)])
        off = c * nacc
        plsc.subcore_barrier()

        def fire(b, j):
            srcb, hsrcb, dstb = b[0], b[1], b[2]
            pltpu.sync_copy(srcr_r.at[s, j], srcb)
            pltpu.sync_copy(dstr_r.at[s, j], dstb)
            for t in range(k_edge // 16):
                hsrcb[pl.ds(t * 16, 16)] = srcb[pl.ds(t * 16, 16)] + off
            cp1 = pltpu.async_copy(htab_r.at[hsrcb], b[3], b[7])
            cp2 = pltpu.async_copy(auxs_r.at[hsrcb], b[4], b[8])
            cp3 = pltpu.async_copy(auxd_r.at[dstb], b[5], b[9])
            return cp1, cp2, cp3

        def drain_compute(b, cps):
            dstb, hrows, srows, drows, wbuf = b[2], b[3], b[4], b[5], b[6]
            for cp in cps:
                cp.wait()

            def edge(k, _):
                t = srows[k] + drows[k]
                w = jnp.exp(jnp.maximum(t, 0.2 * t))
                wbuf[k] = w
                for t8 in range(nt8):
                    head = (c * nt8 + t8) // cph
                    hidx = jnp.full((16,), head, dtype=jnp.int32)
                    ws = lax.gather(
                        w, hidx[:, None],
                        dimension_numbers=lax.GatherDimensionNumbers(
                            offset_dims=(), collapsed_slice_dims=(0,),
                            start_index_map=(0,)),
                        slice_sizes=(1,),
                        mode=lax.GatherScatterMode.PROMISE_IN_BOUNDS)
                    hrows[k, pl.ds(t8 * 16, 16)] = hrows[k, pl.ds(t8 * 16, 16)] * ws
                return 0

            lax.fori_loop(0, k_edge, edge, 0)
            pltpu.sync_copy(hrows, num_sh.at[dstb], add=True)

            @pl.when(c == 0)
            def _():
                pltpu.sync_copy(wbuf, den_sh.at[dstb], add=True)

        def pair(jj, _):
            j0 = 2 * jj
            cps0 = fire(bufs[0], j0)
            cps1 = fire(bufs[1], j0 + 1)
            drain_compute(bufs[0], cps0)
            drain_compute(bufs[1], cps1)
            return 0

        lax.fori_loop(0, ch2 // 2, pair, 0)
        plsc.subcore_barrier()
        pltpu.sync_copy(num_sh.at[pl.ds(r0, rpt)], num_o.at[c, pl.ds(r0, rpt)])

        @pl.when(c == 0)
        def _():
            pltpu.sync_copy(den_sh.at[pl.ds(r0, rpt)], den_o.at[pl.ds(r0, rpt)])

    return kfn(htab, aux_s2, aux_d, srcr, dstr)


def _fold_attn(w, a_s, a_d):
    d, m = w.shape
    c = m // H
    wr = w.reshape(d, H, c)
    ws = jnp.einsum('dhc,hc->dh', wr, a_s)
    wd = jnp.einsum('dhc,hc->dh', wr, a_d)
    return jnp.concatenate([w, ws, wd], axis=1)


def _pad2(a, rows, cols):
    return jnp.pad(a, ((0, rows - a.shape[0]), (0, cols - a.shape[1])))


def _head_expand_mat(c, m_in):
    b = np.zeros((128, m_in), np.float32)
    for h in range(H):
        b[h, h * c:(h + 1) * c] = 1.0
    return jnp.asarray(b)


def _layer_tables(hcat, crow, cw, n, nacc):
    h = hcat[:n, :crow]
    h0 = h[:, :cw]
    h1 = _pad2(h[:, cw:crow], n, cw)
    htab = jnp.concatenate([_pad2(h0, nacc, cw), _pad2(h1, nacc, cw)], axis=0)
    aux_s = _pad2(hcat[:n, crow:crow + H], nacc, 16)
    aux_d = _pad2(hcat[:n, crow + H:crow + 2 * H], nacc, 16)
    aux_s2 = jnp.concatenate([aux_s, aux_s], axis=0)
    return htab, aux_s2, aux_d


def kernel(x, edge_index, W1, as1, ad1, b1, W2, as2, ad2, b2, W3, as3, ad3, b3):
    n, d = x.shape
    e = edge_index.shape[1]
    npad = ((n + 1023) // 1024) * 1024
    nacc = ((n + 8 + NSUB * 8 - 1) // (NSUB * 8)) * (NSUB * 8)

    loops = jnp.arange(n, dtype=jnp.int32)
    src = jnp.concatenate([edge_index[0].astype(jnp.int32), loops])
    dst = jnp.concatenate([edge_index[1].astype(jnp.int32), loops])
    etot = e + n

    def edges_for(k_edge):
        ch2 = -(-etot // (NSUB * k_edge))
        ch2 += ch2 % 2
        ept = NSUB * ch2 * k_edge
        s = jnp.pad(src, (0, ept - etot), constant_values=n).reshape(NSUB, ch2, k_edge)
        t = jnp.pad(dst, (0, ept - etot), constant_values=n).reshape(NSUB, ch2, k_edge)
        return s, t, ch2

    c1, c2, c3 = W1.shape[1] // H, W2.shape[1] // H, W3.shape[1] // H
    crow1, crow2, crow3 = H * c1, H * c2, H * c3
    cw1, cw2, cw3 = crow1 // 2, crow2 // 2, 48
    k1, k2, k3 = 32, 128, 128
    m1p, m2p, m3p = 384, 256, 128

    wa1 = _pad2(_fold_attn(W1, as1, ad1), d, m1p)
    wa2 = _pad2(_fold_attn(W2, as2, ad2), m1p, m2p)
    wa3 = _pad2(_fold_attn(W3, as3, ad3), m2p, m3p)
    bm2 = _head_expand_mat(c1, m1p)
    bm3 = _head_expand_mat(c2, m2p)
    bm4 = _head_expand_mat(c3, m3p)
    b1r = jnp.tile(jnp.pad(b1, (0, m1p - crow1))[None, :], (8, 1))
    b2r = jnp.tile(jnp.pad(b2, (0, m2p - crow2))[None, :], (8, 1))
    b3r = jnp.tile(jnp.pad(b3, (0, m3p - crow3))[None, :], (8, 1))

    xp = jnp.pad(x, ((0, npad - n), (0, 0)))

    def edge_pass(hcat, crow, cw, c_head, k_edge):
        htab, aux_s2, aux_d = _layer_tables(hcat, crow, cw, n, nacc)
        sr, dr, ch2 = edges_for(k_edge)
        num, den = _sc_edge(htab, aux_s2, aux_d, sr, dr,
                            cw=cw, c_head=c_head, nacc=nacc, ch2=ch2,
                            k_edge=k_edge)
        num = jnp.concatenate([num[0], num[1]], axis=1)
        return num, den

    hcat1 = _mm(xp, wa1)
    num1, den1 = edge_pass(hcat1, crow1, cw1, c1, k1)
    num1 = _pad2(num1, npad, m1p)
    den1 = _pad2(den1, npad, 128)
    hcat2 = _div_selu_mm(num1, den1, bm2, b1r, wa2)
    num2, den2 = edge_pass(hcat2, crow2, cw2, c2, k2)
    num2 = _pad2(num2, npad, m2p)
    den2 = _pad2(den2, npad, 128)
    hcat3 = _div_selu_mm(num2, den2, bm3, b2r, wa3)
    num3, den3 = edge_pass(hcat3, crow3, cw3, c3, k3)
    num3 = _pad2(num3, npad, m3p)
    den3 = _pad2(den3, npad, 128)
    out = _final(num3, den3, bm4, b3r)
    return out[:n, :crow3]
